# Initial kernel scaffold; baseline (speedup 1.0000x reference)
import jax
import jax.numpy as jnp
from jax.experimental import pallas as pl


def kernel(h, edge_index_mp, deg_mp, W_self, W_nb1, W_hp, W_nb2, bias, branch_logits, ln_gamma, ln_beta):
    raise NotImplementedError("write your pallas kernel here")


# speedup vs baseline: 4.5582x; 4.5582x over previous
import functools

import jax
import jax.numpy as jnp
from jax import lax
from jax.experimental import pallas as pl
from jax.experimental.pallas import tpu as pltpu
from jax.experimental.pallas import tpu_sc as plsc

_NC = 2
_NS = 16
_CHUNK = 80


def _agg_body(table, src, dst, out, sidx_v, didx_v, rows_v, zbuf, acc, sem,
              *, n_pad, n_edges):
    cid = lax.axis_index("c")
    sid = lax.axis_index("s")

    z16 = jnp.zeros((16,), jnp.float32)

    def zbody(i, _):
        for cc in range(8):
            zbuf[i, pl.ds(cc * 16, 16)] = z16
        return ()

    lax.fori_loop(0, zbuf.shape[0], zbody, ())
    rpt = n_pad // _NS
    zrows = zbuf.shape[0]
    for k in range(rpt // zrows):
        pltpu.sync_copy(zbuf, acc.at[pl.ds(sid * rpt + k * zrows, zrows)])
    plsc.subcore_barrier()

    ept = n_edges // (_NC * _NS)
    wid = sid * _NC + cid
    base = wid * ept
    nchunks = ept // _CHUNK

    def body(i, _):
        off = base + i * _CHUNK
        pltpu.sync_copy(src.at[pl.ds(off, _CHUNK)], sidx_v)
        pltpu.sync_copy(dst.at[pl.ds(off, _CHUNK)], didx_v)
        pltpu.async_copy(table.at[sidx_v], rows_v, sem).wait()
        pltpu.sync_copy(rows_v, acc.at[didx_v], add=True)
        return ()

    lax.fori_loop(0, nchunks, body, ())
    plsc.subcore_barrier()

    pltpu.sync_copy(acc.at[pl.ds(sid * rpt, rpt)],
                    out.at[cid, pl.ds(sid * rpt, rpt)])


def _sc_agg(table, src, dst):
    n_rows, d = table.shape
    n_edges = src.shape[0]
    n_pad = ((n_rows + _NS * 128 - 1) // (_NS * 128)) * (_NS * 128)
    mesh = plsc.VectorSubcoreMesh(core_axis_name="c", subcore_axis_name="s")
    kern = pl.kernel(
        functools.partial(_agg_body, n_pad=n_pad, n_edges=n_edges),
        out_type=jax.ShapeDtypeStruct((_NC, n_pad, d), jnp.float32),
        mesh=mesh,
        scratch_types=[
            pltpu.VMEM((_CHUNK,), jnp.int32),
            pltpu.VMEM((_CHUNK,), jnp.int32),
            pltpu.VMEM((_CHUNK, d), jnp.float32),
            pltpu.VMEM((128, d), jnp.float32),
            pltpu.VMEM_SHARED((n_pad, d), jnp.float32),
            pltpu.SemaphoreType.DMA,
        ],
    )
    return kern(table, src, dst)


def _combine_body(p_ref, deg_ref, out_ref):
    out_ref[...] = (p_ref[0] + p_ref[1]) / deg_ref[...]


def _combine(p, deg2d):
    _, n, d = p.shape
    bn = 1000
    return pl.pallas_call(
        _combine_body,
        grid=(n // bn,),
        in_specs=[
            pl.BlockSpec((2, bn, d), lambda i: (0, i, 0)),
            pl.BlockSpec((bn, 1), lambda i: (i, 0)),
        ],
        out_specs=pl.BlockSpec((bn, d), lambda i: (i, 0)),
        out_shape=jax.ShapeDtypeStruct((n, d), jnp.float32),
    )(p, deg2d)


def _final_body(h_ref, nb1_ref, q_ref, deg_ref, ws, w1, whp, w2,
                b_ref, lg_ref, g_ref, bt_ref, out_ref):
    s = 2.0 * jax.nn.sigmoid(lg_ref[...])
    hb = h_ref[...]
    n1 = nb1_ref[...]
    n2 = (q_ref[0] + q_ref[1]) / deg_ref[...]
    hp = hb - n1
    dot = functools.partial(jnp.dot, preferred_element_type=jnp.float32,
                            precision=lax.Precision.HIGHEST)
    z = (s[0:1, :] * dot(hb, ws[...])
         + s[1:2, :] * dot(n1, w1[...])
         + s[2:3, :] * dot(hp, whp[...])
         + s[3:4, :] * dot(n2, w2[...])
         + b_ref[...])
    mu = jnp.mean(z, axis=-1, keepdims=True)
    zc = z - mu
    var = jnp.mean(zc * zc, axis=-1, keepdims=True)
    out_ref[...] = zc * lax.rsqrt(var + 1e-5) * g_ref[...] + bt_ref[...]


def _final(h, nb1, q, deg2d, wst, w1t, whpt, w2t, bias2d, lg, gamma2d, beta2d):
    n, d = h.shape
    bn = 1000
    row = lambda i: (i, 0)
    full = lambda i: (0, 0)
    return pl.pallas_call(
        _final_body,
        grid=(n // bn,),
        in_specs=[
            pl.BlockSpec((bn, d), row),
            pl.BlockSpec((bn, d), row),
            pl.BlockSpec((2, bn, d), lambda i: (0, i, 0)),
            pl.BlockSpec((bn, 1), row),
            pl.BlockSpec((d, d), full),
            pl.BlockSpec((d, d), full),
            pl.BlockSpec((d, d), full),
            pl.BlockSpec((d, d), full),
            pl.BlockSpec((1, d), full),
            pl.BlockSpec((4, d), full),
            pl.BlockSpec((1, d), full),
            pl.BlockSpec((1, d), full),
        ],
        out_specs=pl.BlockSpec((bn, d), row),
        out_shape=jax.ShapeDtypeStruct((n, d), jnp.float32),
    )(h, nb1, q, deg2d, wst, w1t, whpt, w2t, bias2d, lg, gamma2d, beta2d)


def kernel(h, edge_index_mp, deg_mp, W_self, W_nb1, W_hp, W_nb2, bias,
           branch_logits, ln_gamma, ln_beta):
    n, d = h.shape
    src = edge_index_mp[0].astype(jnp.int32)
    dst = edge_index_mp[1].astype(jnp.int32)
    deg2d = deg_mp.reshape(n, 1)

    p = _sc_agg(h, src, dst)
    nb1 = _combine(p, deg2d)
    q = _sc_agg(nb1, src, dst)

    lg = jnp.broadcast_to(branch_logits[:, None], (4, d))
    return _final(h, nb1, q, deg2d,
                  W_self.T, W_nb1.T, W_hp.T, W_nb2.T,
                  bias.reshape(1, d), lg,
                  ln_gamma.reshape(1, d), ln_beta.reshape(1, d))
